# baseline (speedup 1.0000x reference)
import functools

import jax
import jax.numpy as jnp
from jax import lax
from jax.experimental import pallas as pl
from jax.experimental.pallas import tpu as pltpu
from jax.experimental.pallas import tpu_sc as plsc

N_NODES = 10000
N_EDGES = 320000
LANES = 16


def _table_body(x_ref, wd_ref, p_ref):
    p_ref[...] = jnp.exp(-lax.dot_general(
        wd_ref[...], x_ref[...], (((0,), (1,)), ((), ())),
        preferred_element_type=jnp.float32))


NBLK = 79
EDGES_W = NBLK * 128


def _sc_body(tab_hbm, ei_hbm, out_hbm, tab_v, row_v, col_v, os_v, ot_v, sem,
             *, num_cores):
    wid = lax.axis_index("s") * num_cores + lax.axis_index("c")
    bstart = 78 * wid + jnp.maximum(wid - 28, 0)
    ebase = bstart * 128
    h1 = pltpu.async_copy(tab_hbm, tab_v, sem)
    h2 = pltpu.async_copy(ei_hbm.at[0, pl.ds(ebase, EDGES_W)], row_v, sem)
    h3 = pltpu.async_copy(ei_hbm.at[1, pl.ds(ebase, EDGES_W)], col_v, sem)
    h1.wait()
    h2.wait()
    h3.wait()

    @plsc.parallel_loop(0, NBLK, step=1, unroll=4)
    def blk(b):
        ein = b * 128
        for g in range(128 // LANES):
            r = row_v[pl.ds(ein + g * LANES, LANES)]
            c = col_v[pl.ds(ein + g * LANES, LANES)]
            eu = plsc.load_gather(tab_v, [r]) * \
                plsc.load_gather(tab_v, [c + N_NODES])
            ew = plsc.load_gather(tab_v, [r + 2 * N_NODES]) * \
                plsc.load_gather(tab_v, [c + 3 * N_NODES])
            s = 1.0 / (1.0 + eu)
            t = 1.0 / (1.0 + ew)
            os_v[b, 0, pl.ds(g * LANES, LANES)] = s
            os_v[b, 1, pl.ds(g * LANES, LANES)] = -s
            ot_v[b, 0, pl.ds(g * LANES, LANES)] = -t
            ot_v[b, 1, pl.ds(g * LANES, LANES)] = t

    h4 = pltpu.async_copy(os_v, out_hbm.at[0, pl.ds(bstart, NBLK)], sem)
    h5 = pltpu.async_copy(ot_v, out_hbm.at[1, pl.ds(bstart, NBLK)], sem)
    h4.wait()
    h5.wait()


def kernel(x, edge_index, W):
    Wr, Wc = W[:, :128], W[:, 128:]
    wd = jnp.stack([Wr[1] - Wr[0], Wc[1] - Wc[0],
                    Wr[2] - Wr[3], Wc[2] - Wc[3]], axis=1)
    wd_pad = jnp.pad(wd, ((0, 0), (0, 4)))

    p8 = pl.pallas_call(
        _table_body,
        out_shape=jax.ShapeDtypeStruct((8, N_NODES), jnp.float32),
    )(x, wd_pad)
    tab = p8[:4].reshape(-1)

    info = plsc.get_sparse_core_info()

    mesh = plsc.VectorSubcoreMesh(core_axis_name="c", subcore_axis_name="s")
    sc = pl.kernel(
        functools.partial(_sc_body, num_cores=info.num_cores),
        out_type=jax.ShapeDtypeStruct((2, N_EDGES // 128, 2, 128),
                                      jnp.float32),
        mesh=mesh,
        compiler_params=pltpu.CompilerParams(needs_layout_passes=False,
                                             use_tc_tiling_on_sc=False,
                                             skip_device_barrier=True),
        scratch_types=[
            pltpu.VMEM((N_NODES * 4,), jnp.float32),
            pltpu.VMEM((EDGES_W,), jnp.int32),
            pltpu.VMEM((EDGES_W,), jnp.int32),
            pltpu.VMEM((NBLK, 2, 128), jnp.float32),
            pltpu.VMEM((NBLK, 2, 128), jnp.float32),
            pltpu.SemaphoreType.DMA,
        ],
    )
    st4 = sc(tab, edge_index)
    return st4.transpose(1, 3, 0, 2).reshape(N_EDGES, 2, 2)
